# baseline (speedup 1.0000x reference)
import jax
import jax.numpy as jnp
from jax.experimental import pallas as pl
from jax.experimental.pallas import tpu as pltpu

_B, _T, _P, _D, _K, _TEMP = 256, 128, 128, 64, 4, 8.0
_PF = float(_P)


def _ring_kernel(x_ref, wi_ref, biT_ref, proc_w_ref, proc_bT_ref,
                 gate_w_ref, gate_bT_ref, wo_ref, boT_ref,
                 dest_ref, cs_ref, p1_ref, p2_ref,
                 out_ref, mem_ref):
    B, T, P, D = _B, _T, _P, _D

    mem_ref[...] = jnp.zeros((P, D, B), jnp.float32)

    wi = wi_ref[...]
    biT = biT_ref[...]
    cs_row = cs_ref[...]
    cs1 = cs_row[:, 0:1]
    cs2 = cs_row[:, 1:2]
    dest1T = dest_ref[:, 0:1]
    dest2T = dest_ref[:, 1:2]
    proc_w = proc_w_ref[...]
    proc_bT = proc_bT_ref[...]
    gate_w = gate_w_ref[...]
    gate_bT = gate_bT_ref[...]
    wo = wo_ref[...]
    boT = boT_ref[...]
    piotaT = jax.lax.broadcasted_iota(jnp.int32, (P, B), 0).astype(jnp.float32)

    def fmod_p(v):
        return v - _PF * jnp.floor(v * (1.0 / _PF))

    def gauss(p):
        base = jnp.clip(jnp.floor(p), 0.0, _PF - 1.0)
        delta = fmod_p(piotaT - p + _PF * 0.5) - _PF * 0.5
        db = fmod_p(piotaT - base + _PF * 0.5) - _PF * 0.5
        mask = jnp.abs(db) <= float(_K)
        logits = jnp.where(mask, delta * delta * (-1.0 / _TEMP), -jnp.inf)
        m = jnp.max(logits, axis=0, keepdims=True)
        e = jnp.where(mask, jnp.exp(logits - m), 0.0)
        denom = jnp.sum(e, axis=0, keepdims=True)
        return e / denom, base

    def body(t, carry):
        hT, p1, p2 = carry
        xt = x_ref[pl.ds(t * 8, 8), :]
        ivT = jax.lax.dot_general(
            wi, xt, (((0,), (0,)), ((), ())),
            preferred_element_type=jnp.float32) + biT
        g1T, base1 = gauss(p1)
        g2T, base2 = gauss(p2)
        gcombT = cs1 * g1T + cs2 * g2T
        mem = mem_ref[...]
        crT = jnp.sum(mem * gcombT[:, None, :], axis=0)
        s1T = jnp.tanh(ivT + crT + hT)
        sT = jnp.tanh(
            jax.lax.dot_general(
                proc_w, s1T, (((0,), (0,)), ((), ())),
                preferred_element_type=jnp.float32) + proc_bT
        )
        mem_ref[...] = mem + g1T[:, None, :] * sT[None, :, :]
        jlT = jax.lax.dot_general(
            gate_w, sT, (((0,), (0,)), ((), ())),
            preferred_element_type=jnp.float32) + gate_bT
        obT = jax.lax.dot_general(
            wo, sT, (((0,), (0,)), ((), ())),
            preferred_element_type=jnp.float32) + boT
        out_ref[t] = obT
        jt1 = jnp.sum(jnp.where(piotaT == base1, dest1T, 0.0), axis=0,
                      keepdims=True)
        jt2 = jnp.sum(jnp.where(piotaT == base2, dest2T, 0.0), axis=0,
                      keepdims=True)
        p1n = jnp.where(jlT[0:1, :] > 0.0, jt1, fmod_p(p1 + 1.0))
        p2n = jnp.where(jlT[1:2, :] > 0.0, jt2, fmod_p(p2 + 1.0))
        return (sT, p1n, p2n)

    h0 = jnp.zeros((D, B), jnp.float32)
    jax.lax.fori_loop(0, T, body, (h0, p1_ref[...], p2_ref[...]))


def kernel(x, input_proj_w, input_proj_b, output_proj_w, output_proj_b,
           pointer1_destinations, pointer1_gate_w, pointer1_gate_b,
           context1_strength, pointer2_destinations, pointer2_gate_w,
           pointer2_gate_b, context2_strength, proc_w, proc_b,
           p1_init, p2_init):
    B, T, P, D = _B, _T, _P, _D
    x_t8b = jnp.transpose(x, (1, 2, 0)).reshape(T * 8, B)
    cs = jnp.stack([jax.nn.sigmoid(context1_strength),
                    jax.nn.sigmoid(context2_strength)]).reshape(1, 2)
    gate_w = jnp.concatenate([pointer1_gate_w, pointer2_gate_w], axis=1)
    gate_bT = jnp.concatenate([pointer1_gate_b, pointer2_gate_b]).reshape(2, 1)
    dest = jnp.stack([pointer1_destinations, pointer2_destinations],
                     axis=1)

    outT = pl.pallas_call(
        _ring_kernel,
        out_shape=jax.ShapeDtypeStruct((T, 8, B), jnp.float32),
        scratch_shapes=[
            pltpu.VMEM((P, D, B), jnp.float32),
        ],
    )(
        x_t8b,
        input_proj_w, input_proj_b.reshape(D, 1),
        proc_w, proc_b.reshape(D, 1),
        gate_w, gate_bT,
        output_proj_w, output_proj_b.reshape(8, 1),
        dest, cs,
        p1_init.reshape(1, B), p2_init.reshape(1, B),
    )
    return jnp.transpose(outT, (2, 0, 1))


# speedup vs baseline: 1.2424x; 1.2424x over previous
import jax
import jax.numpy as jnp
from jax.experimental import pallas as pl
from jax.experimental.pallas import tpu as pltpu

_B, _T, _P, _D, _K, _TEMP = 256, 128, 128, 64, 4, 8.0
_PF = float(_P)


def _ring_kernel(x_ref, wi_ref, biT_ref, proc_w_ref, proc_bT_ref,
                 gate_w_ref, gate_bT_ref, wo_ref, boT_ref,
                 dest_ref, cs_ref, p1_ref, p2_ref,
                 out_ref, mem_ref):
    B, T, P, D = _B, _T, _P, _D

    mem_ref[...] = jnp.zeros((D, P, B), jnp.float32)

    wi = wi_ref[...]
    biT = biT_ref[...]
    cs_row = cs_ref[...]
    cs1 = cs_row[:, 0:1]
    cs2 = cs_row[:, 1:2]
    dest1T = dest_ref[:, 0:1]
    dest2T = dest_ref[:, 1:2]
    proc_w = proc_w_ref[...]
    proc_bT = proc_bT_ref[...]
    gate_w = gate_w_ref[...]
    gate_bT = gate_bT_ref[...]
    wo = wo_ref[...]
    boT = boT_ref[...]
    piotaT = jax.lax.broadcasted_iota(jnp.int32, (P, B), 0).astype(jnp.float32)

    def fmod_p(v):
        return v - _PF * jnp.floor(v * (1.0 / _PF))

    def gauss(p):
        base = jnp.clip(jnp.floor(p), 0.0, _PF - 1.0)
        delta = fmod_p(piotaT - p + _PF * 0.5) - _PF * 0.5
        db = fmod_p(piotaT - base + _PF * 0.5) - _PF * 0.5
        mask = jnp.abs(db) <= float(_K)
        logits = jnp.where(mask, delta * delta * (-1.0 / _TEMP), -jnp.inf)
        m = jnp.max(logits, axis=0, keepdims=True)
        e = jnp.where(mask, jnp.exp(logits - m), 0.0)
        denom = jnp.sum(e, axis=0, keepdims=True)
        return e / denom, db

    def body(t, carry):
        hT, p1, p2 = carry
        xt = x_ref[pl.ds(t * 8, 8), :]
        ivT = jax.lax.dot_general(
            wi, xt, (((0,), (0,)), ((), ())),
            preferred_element_type=jnp.float32) + biT
        g1T, db1 = gauss(p1)
        g2T, db2 = gauss(p2)
        gcombT = cs1 * g1T + cs2 * g2T
        mem = mem_ref[...]
        crT = jnp.sum(mem * gcombT[None, :, :], axis=1)
        s1T = jnp.tanh(ivT + crT + hT)
        sT = jnp.tanh(
            jax.lax.dot_general(
                proc_w, s1T, (((0,), (0,)), ((), ())),
                preferred_element_type=jnp.float32) + proc_bT
        )
        mem_ref[...] = mem + g1T[None, :, :] * sT[:, None, :]
        jlT = jax.lax.dot_general(
            gate_w, sT, (((0,), (0,)), ((), ())),
            preferred_element_type=jnp.float32) + gate_bT
        obT = jax.lax.dot_general(
            wo, sT, (((0,), (0,)), ((), ())),
            preferred_element_type=jnp.float32) + boT
        out_ref[t] = obT
        jt1 = jnp.sum(jnp.where(db1 == 0.0, dest1T, 0.0), axis=0,
                      keepdims=True)
        jt2 = jnp.sum(jnp.where(db2 == 0.0, dest2T, 0.0), axis=0,
                      keepdims=True)
        p1n = jnp.where(jlT[0:1, :] > 0.0, jt1, fmod_p(p1 + 1.0))
        p2n = jnp.where(jlT[1:2, :] > 0.0, jt2, fmod_p(p2 + 1.0))
        return (sT, p1n, p2n)

    h0 = jnp.zeros((D, B), jnp.float32)
    jax.lax.fori_loop(0, T, body, (h0, p1_ref[...], p2_ref[...]))


def kernel(x, input_proj_w, input_proj_b, output_proj_w, output_proj_b,
           pointer1_destinations, pointer1_gate_w, pointer1_gate_b,
           context1_strength, pointer2_destinations, pointer2_gate_w,
           pointer2_gate_b, context2_strength, proc_w, proc_b,
           p1_init, p2_init):
    B, T, P, D = _B, _T, _P, _D
    x_t8b = jnp.transpose(x, (1, 2, 0)).reshape(T * 8, B)
    cs = jnp.stack([jax.nn.sigmoid(context1_strength),
                    jax.nn.sigmoid(context2_strength)]).reshape(1, 2)
    gate_w = jnp.concatenate([pointer1_gate_w, pointer2_gate_w], axis=1)
    gate_bT = jnp.concatenate([pointer1_gate_b, pointer2_gate_b]).reshape(2, 1)
    dest = jnp.stack([pointer1_destinations, pointer2_destinations],
                     axis=1)

    outT = pl.pallas_call(
        _ring_kernel,
        out_shape=jax.ShapeDtypeStruct((T, 8, B), jnp.float32),
        scratch_shapes=[
            pltpu.VMEM((D, P, B), jnp.float32),
        ],
    )(
        x_t8b,
        input_proj_w, input_proj_b.reshape(D, 1),
        proc_w, proc_b.reshape(D, 1),
        gate_w, gate_bT,
        output_proj_w, output_proj_b.reshape(8, 1),
        dest, cs,
        p1_init.reshape(1, B), p2_init.reshape(1, B),
    )
    return jnp.transpose(outT, (2, 0, 1))


# speedup vs baseline: 1.2540x; 1.0094x over previous
import jax
import jax.numpy as jnp
from jax.experimental import pallas as pl
from jax.experimental.pallas import tpu as pltpu

_B, _T, _P, _D, _K, _TEMP = 256, 128, 128, 64, 4, 8.0
_PF = float(_P)


def _ring_kernel(x_ref, wi_ref, biT_ref, proc_w_ref, proc_bT_ref,
                 gate_w_ref, gate_bT_ref, wo_ref, boT_ref,
                 dest_ref, cs_ref, p1_ref, p2_ref,
                 out_ref, mem_ref):
    B, T, P, D = _B, _T, _P, _D

    mem_ref[...] = jnp.zeros((D, P, B), jnp.float32)

    wi = wi_ref[...]
    biT = biT_ref[...]
    cs_row = cs_ref[...]
    cs1 = cs_row[:, 0:1]
    cs2 = cs_row[:, 1:2]
    dest1T = dest_ref[:, 0:1]
    dest2T = dest_ref[:, 1:2]
    proc_w = proc_w_ref[...]
    proc_bT = proc_bT_ref[...]
    gate_w = gate_w_ref[...]
    gate_bT = gate_bT_ref[...]
    wo = wo_ref[...]
    boT = boT_ref[...]
    piotaT = jax.lax.broadcasted_iota(jnp.int32, (P, B), 0).astype(jnp.float32)

    def fmod_p(v):
        return v - _PF * jnp.floor(v * (1.0 / _PF))

    def gauss(p):
        base = jnp.clip(jnp.floor(p), 0.0, _PF - 1.0)
        delta = fmod_p(piotaT - p + _PF * 0.5) - _PF * 0.5
        db = fmod_p(piotaT - base + _PF * 0.5) - _PF * 0.5
        mask = jnp.abs(db) <= float(_K)
        logits = jnp.where(mask, delta * delta * (-1.0 / _TEMP), -jnp.inf)
        m = jnp.max(logits, axis=0, keepdims=True)
        e = jnp.where(mask, jnp.exp(logits - m), 0.0)
        denom = jnp.sum(e, axis=0, keepdims=True)
        return e / denom, base

    def body(t, carry):
        hT, p1, p2 = carry
        xt = x_ref[pl.ds(t * 8, 8), :]
        ivT = jax.lax.dot_general(
            wi, xt, (((0,), (0,)), ((), ())),
            preferred_element_type=jnp.float32) + biT
        g1T, base1 = gauss(p1)
        g2T, base2 = gauss(p2)
        gcombT = cs1 * g1T + cs2 * g2T
        mem = mem_ref[...]
        crT = jnp.sum(mem * gcombT[None, :, :], axis=1)
        s1T = jnp.tanh(ivT + crT + hT)
        sT = jnp.tanh(
            jax.lax.dot_general(
                proc_w, s1T, (((0,), (0,)), ((), ())),
                preferred_element_type=jnp.float32) + proc_bT
        )
        mem_ref[...] = mem + g1T[None, :, :] * sT[:, None, :]
        jlT = jax.lax.dot_general(
            gate_w, sT, (((0,), (0,)), ((), ())),
            preferred_element_type=jnp.float32) + gate_bT
        obT = jax.lax.dot_general(
            wo, sT, (((0,), (0,)), ((), ())),
            preferred_element_type=jnp.float32) + boT
        out_ref[t] = obT
        jt1 = jnp.sum(jnp.where(piotaT == base1, dest1T, 0.0), axis=0,
                      keepdims=True)
        jt2 = jnp.sum(jnp.where(piotaT == base2, dest2T, 0.0), axis=0,
                      keepdims=True)
        p1n = jnp.where(jlT[0:1, :] > 0.0, jt1, fmod_p(p1 + 1.0))
        p2n = jnp.where(jlT[1:2, :] > 0.0, jt2, fmod_p(p2 + 1.0))
        return (sT, p1n, p2n)

    h0 = jnp.zeros((D, B), jnp.float32)
    jax.lax.fori_loop(0, T, body, (h0, p1_ref[...], p2_ref[...]))


def kernel(x, input_proj_w, input_proj_b, output_proj_w, output_proj_b,
           pointer1_destinations, pointer1_gate_w, pointer1_gate_b,
           context1_strength, pointer2_destinations, pointer2_gate_w,
           pointer2_gate_b, context2_strength, proc_w, proc_b,
           p1_init, p2_init):
    B, T, P, D = _B, _T, _P, _D
    x_t8b = jnp.transpose(x, (1, 2, 0)).reshape(T * 8, B)
    cs = jnp.stack([jax.nn.sigmoid(context1_strength),
                    jax.nn.sigmoid(context2_strength)]).reshape(1, 2)
    gate_w = jnp.concatenate([pointer1_gate_w, pointer2_gate_w], axis=1)
    gate_bT = jnp.concatenate([pointer1_gate_b, pointer2_gate_b]).reshape(2, 1)
    dest = jnp.stack([pointer1_destinations, pointer2_destinations],
                     axis=1)

    outT = pl.pallas_call(
        _ring_kernel,
        out_shape=jax.ShapeDtypeStruct((T, 8, B), jnp.float32),
        scratch_shapes=[
            pltpu.VMEM((D, P, B), jnp.float32),
        ],
    )(
        x_t8b,
        input_proj_w, input_proj_b.reshape(D, 1),
        proc_w, proc_b.reshape(D, 1),
        gate_w, gate_bT,
        output_proj_w, output_proj_b.reshape(8, 1),
        dest, cs,
        p1_init.reshape(1, B), p2_init.reshape(1, B),
    )
    return jnp.transpose(outT, (2, 0, 1))
